# Initial kernel scaffold; baseline (speedup 1.0000x reference)
import jax
import jax.numpy as jnp
from jax.experimental import pallas as pl


def kernel(feat, edge_index, W, b, W_loop):
    raise NotImplementedError("write your pallas kernel here")


# speedup vs baseline: 2.6726x; 2.6726x over previous
import functools

import jax
import jax.numpy as jnp
from jax import lax
from jax.experimental import pallas as pl
from jax.experimental.pallas import tpu as pltpu
from jax.experimental.pallas import tpu_sc as plsc

F = 128
NC, NS = 2, 16
NW = NC * NS
CH = 128
LANES = 16


def _hist_body(sidx, didx, degs_out, degd_out, idx_v, ones_v, zero_v, hs, hd,
               *, chunks, npad):
  rpt = npad // NS
  c = lax.axis_index("c")
  s = lax.axis_index("s")
  w = s * NC + c

  def fill(i, _):
    ones_v[pl.ds(i * LANES, LANES)] = jnp.full((LANES,), 1.0, jnp.float32)
    return 0
  lax.fori_loop(0, CH // LANES, fill, 0)

  def zfill(i, _):
    zero_v[pl.ds(i * LANES, LANES)] = jnp.zeros((LANES,), jnp.float32)
    return 0
  lax.fori_loop(0, rpt // LANES, zfill, 0)

  pltpu.sync_copy(zero_v, hs.at[pl.ds(s * rpt, rpt)])
  pltpu.sync_copy(zero_v, hd.at[pl.ds(s * rpt, rpt)])
  plsc.subcore_barrier()

  pltpu.sync_copy(sidx.at[w], idx_v)

  def sbody(j, _):
    pltpu.sync_copy(ones_v, hs.at[idx_v.at[j]], add=True)
    return 0
  lax.fori_loop(0, chunks, sbody, 0)

  pltpu.sync_copy(didx.at[w], idx_v)

  def dbody(j, _):
    pltpu.sync_copy(ones_v, hd.at[idx_v.at[j]], add=True)
    return 0
  lax.fori_loop(0, chunks, dbody, 0)

  plsc.subcore_barrier()
  pltpu.sync_copy(hs.at[pl.ds(s * rpt, rpt)], degs_out.at[c, pl.ds(s * rpt, rpt)])
  pltpu.sync_copy(hd.at[pl.ds(s * rpt, rpt)], degd_out.at[c, pl.ds(s * rpt, rpt)])


GRP = 8


def _agg_body(feat_s, eidx, zrows, acc_out, idx_v, rows, acc, gsem,
              *, chunks, npad):
  rpt = npad // NS
  c = lax.axis_index("c")
  s = lax.axis_index("s")
  w = s * NC + c

  pltpu.sync_copy(zrows, rows.at[0])
  for r in range(rpt // CH):
    pltpu.sync_copy(rows.at[0], acc.at[pl.ds(s * rpt + r * CH, CH)])
  plsc.subcore_barrier()

  def body(g, _):
    pltpu.sync_copy(eidx.at[w, pl.ds(g * GRP, GRP)], idx_v)
    for k in range(GRP):
      pltpu.async_copy(feat_s.at[idx_v.at[k, 0]], rows.at[0], gsem).wait()
      pltpu.sync_copy(rows.at[0], acc.at[idx_v.at[k, 1]], add=True)
    return 0
  lax.fori_loop(0, chunks // GRP, body, 0)

  plsc.subcore_barrier()
  pltpu.sync_copy(acc.at[pl.ds(s * rpt, rpt)], acc_out.at[c, pl.ds(s * rpt, rpt)])


def _scale_body(deg_ref, feat_ref, out_ref):
  d = deg_ref[:, 0] + deg_ref[:, 1]
  norm = lax.rsqrt(jnp.maximum(d, 1.0))
  out_ref[...] = feat_ref[...] * norm[:, None]


def _final_body(acc_ref, degd_ref, feat_ref, w_ref, wl_ref, b_ref, out_ref):
  agg = acc_ref[0] + acc_ref[1]
  rst = jnp.dot(agg, w_ref[...], preferred_element_type=jnp.float32)
  d = degd_ref[:, 0] + degd_ref[:, 1]
  norm = lax.rsqrt(jnp.maximum(d, 1.0))
  loop_msg = jnp.dot(feat_ref[...], wl_ref[...], preferred_element_type=jnp.float32)
  out_ref[...] = rst * norm[:, None] + b_ref[0] + loop_msg


def kernel(feat, edge_index, W, b, W_loop):
  n = feat.shape[0]
  e = edge_index.shape[1]
  npad = ((n + NS * CH) // (NS * CH)) * (NS * CH)
  grp_e = NW * CH * GRP
  ept = ((e + grp_e - 1) // grp_e) * (CH * GRP)
  chunks = ept // CH
  epad = ept * NW

  src = edge_index[0].astype(jnp.int32)
  dst = edge_index[1].astype(jnp.int32)
  src_p = jnp.concatenate(
      [src, jnp.full((epad - e,), n, jnp.int32)]).reshape(NW, chunks, CH)
  dst_p = jnp.concatenate(
      [dst, jnp.full((epad - e,), n, jnp.int32)]).reshape(NW, chunks, CH)
  feat_pad = jnp.pad(feat, ((0, npad - n), (0, 0)))

  mesh = plsc.VectorSubcoreMesh(core_axis_name="c", subcore_axis_name="s",
                                num_cores=NC, num_subcores=NS)
  rpt = npad // NS

  hist = pl.kernel(
      functools.partial(_hist_body, chunks=chunks, npad=npad),
      out_type=(jax.ShapeDtypeStruct((NC, npad), jnp.float32),
                jax.ShapeDtypeStruct((NC, npad), jnp.float32)),
      mesh=mesh,
      scratch_types=[
          pltpu.VMEM((chunks, CH), jnp.int32),
          pltpu.VMEM((CH,), jnp.float32),
          pltpu.VMEM((rpt,), jnp.float32),
          pltpu.VMEM_SHARED((npad,), jnp.float32),
          pltpu.VMEM_SHARED((npad,), jnp.float32),
      ],
  )
  degs, degd = hist(src_p, dst_p)

  scale = pl.pallas_call(
      _scale_body,
      grid=(npad // 1024,),
      in_specs=[
          pl.BlockSpec((1024, NC), lambda i: (i, 0)),
          pl.BlockSpec((1024, F), lambda i: (i, 0)),
      ],
      out_specs=pl.BlockSpec((1024, F), lambda i: (i, 0)),
      out_shape=jax.ShapeDtypeStruct((npad, F), jnp.float32),
  )
  feat_scaled = scale(jnp.transpose(degs), feat_pad)

  eidx = jnp.stack([src_p, dst_p], axis=2)
  agg = pl.kernel(
      functools.partial(_agg_body, chunks=chunks, npad=npad),
      out_type=jax.ShapeDtypeStruct((NC, npad, F), jnp.float32),
      mesh=mesh,
      scratch_types=[
          pltpu.VMEM((GRP, 2, CH), jnp.int32),
          pltpu.VMEM((2, CH, F), jnp.float32),
          pltpu.VMEM_SHARED((npad, F), jnp.float32),
          pltpu.SemaphoreType.DMA,
      ],
  )
  acc = agg(feat_scaled, eidx, jnp.zeros((CH, F), jnp.float32))

  blk = 512
  final = pl.pallas_call(
      _final_body,
      grid=(npad // blk,),
      in_specs=[
          pl.BlockSpec((NC, blk, F), lambda i: (0, i, 0)),
          pl.BlockSpec((blk, NC), lambda i: (i, 0)),
          pl.BlockSpec((blk, F), lambda i: (i, 0)),
          pl.BlockSpec((F, F), lambda i: (0, 0)),
          pl.BlockSpec((F, F), lambda i: (0, 0)),
          pl.BlockSpec((1, F), lambda i: (0, 0)),
      ],
      out_specs=pl.BlockSpec((blk, F), lambda i: (i, 0)),
      out_shape=jax.ShapeDtypeStruct((npad, F), jnp.float32),
  )
  out = final(acc, jnp.transpose(degd), feat_pad, W, W_loop,
              b.reshape(1, F))
  return out[:n]


# speedup vs baseline: 2.9299x; 1.0963x over previous
import functools

import jax
import jax.numpy as jnp
from jax import lax
from jax.experimental import pallas as pl
from jax.experimental.pallas import tpu as pltpu
from jax.experimental.pallas import tpu_sc as plsc

F = 128
NC, NS = 2, 16
NW = NC * NS
CH = 128
LANES = 16


def _hist_body(sidx, didx, degs_out, degd_out, idx_v, ones_v, zero_v, hs, hd,
               *, chunks, npad):
  rpt = npad // NS
  c = lax.axis_index("c")
  s = lax.axis_index("s")
  w = s * NC + c

  def fill(i, _):
    ones_v[pl.ds(i * LANES, LANES)] = jnp.full((LANES,), 1.0, jnp.float32)
    return 0
  lax.fori_loop(0, CH // LANES, fill, 0)

  def zfill(i, _):
    zero_v[pl.ds(i * LANES, LANES)] = jnp.zeros((LANES,), jnp.float32)
    return 0
  lax.fori_loop(0, rpt // LANES, zfill, 0)

  pltpu.sync_copy(zero_v, hs.at[pl.ds(s * rpt, rpt)])
  pltpu.sync_copy(zero_v, hd.at[pl.ds(s * rpt, rpt)])
  plsc.subcore_barrier()

  pltpu.sync_copy(sidx.at[w], idx_v)

  def sbody(j, _):
    pltpu.sync_copy(ones_v, hs.at[idx_v.at[j]], add=True)
    return 0
  lax.fori_loop(0, chunks, sbody, 0)

  pltpu.sync_copy(didx.at[w], idx_v)

  def dbody(j, _):
    pltpu.sync_copy(ones_v, hd.at[idx_v.at[j]], add=True)
    return 0
  lax.fori_loop(0, chunks, dbody, 0)

  plsc.subcore_barrier()
  pltpu.sync_copy(hs.at[pl.ds(s * rpt, rpt)], degs_out.at[c, pl.ds(s * rpt, rpt)])
  pltpu.sync_copy(hd.at[pl.ds(s * rpt, rpt)], degd_out.at[c, pl.ds(s * rpt, rpt)])


NBUF = 2
G = 16


def _agg_body(feat_s, eidx, zrows, acc_out, idx_v, rows, acc, sems,
              *, chunks, npad):
  rpt = npad // NS
  c = lax.axis_index("c")
  s = lax.axis_index("s")
  w = s * NC + c
  ngr = chunks // G

  def load_idx(g):
    return pltpu.async_copy(eidx.at[w, pl.ds(g * G, G)], idx_v.at[g % 2],
                            sems.at[NBUF + g % 2])

  icp0 = load_idx(0)
  pltpu.sync_copy(zrows, rows.at[0])
  for r in range(rpt // CH):
    pltpu.sync_copy(rows.at[0], acc.at[pl.ds(s * rpt + r * CH, CH)])
  icp0.wait()
  plsc.subcore_barrier()

  icps = {}
  idx_ready = {0}
  cps = [pltpu.async_copy(feat_s.at[idx_v.at[0, k, 0]], rows.at[k],
                          sems.at[k]) for k in range(NBUF)]
  for j in range(chunks):
    b = j % NBUF
    if j % G == 0 and j // G + 1 < ngr:
      icps[j // G + 1] = load_idx(j // G + 1)
    cps[b].wait()
    pltpu.sync_copy(rows.at[b], acc.at[idx_v.at[(j // G) % 2, j % G, 1]],
                    add=True)
    nxt = j + NBUF
    if nxt < chunks:
      gi = nxt // G
      if gi not in idx_ready:
        icps[gi].wait()
        idx_ready.add(gi)
      cps[b] = pltpu.async_copy(feat_s.at[idx_v.at[gi % 2, nxt % G, 0]],
                                rows.at[b], sems.at[b])

  plsc.subcore_barrier()
  pltpu.sync_copy(acc.at[pl.ds(s * rpt, rpt)], acc_out.at[c, pl.ds(s * rpt, rpt)])


def _scale_body(deg_ref, feat_ref, out_ref):
  d = deg_ref[:, 0] + deg_ref[:, 1]
  norm = lax.rsqrt(jnp.maximum(d, 1.0))
  out_ref[...] = feat_ref[...] * norm[:, None]


def _final_body(acc_ref, degd_ref, feat_ref, w_ref, wl_ref, b_ref, out_ref):
  agg = acc_ref[0] + acc_ref[1]
  rst = jnp.dot(agg, w_ref[...], preferred_element_type=jnp.float32)
  d = degd_ref[:, 0] + degd_ref[:, 1]
  norm = lax.rsqrt(jnp.maximum(d, 1.0))
  loop_msg = jnp.dot(feat_ref[...], wl_ref[...], preferred_element_type=jnp.float32)
  out_ref[...] = rst * norm[:, None] + b_ref[0] + loop_msg


def kernel(feat, edge_index, W, b, W_loop):
  n = feat.shape[0]
  e = edge_index.shape[1]
  npad = ((n + NS * CH) // (NS * CH)) * (NS * CH)
  grp_e = NW * CH * G
  ept = ((e + grp_e - 1) // grp_e) * (CH * G)
  chunks = ept // CH
  epad = ept * NW

  src = edge_index[0].astype(jnp.int32)
  dst = edge_index[1].astype(jnp.int32)
  src_p = jnp.concatenate(
      [src, jnp.full((epad - e,), n, jnp.int32)]).reshape(NW, chunks, CH)
  dst_p = jnp.concatenate(
      [dst, jnp.full((epad - e,), n, jnp.int32)]).reshape(NW, chunks, CH)
  feat_pad = jnp.pad(feat, ((0, npad - n), (0, 0)))

  mesh = plsc.VectorSubcoreMesh(core_axis_name="c", subcore_axis_name="s",
                                num_cores=NC, num_subcores=NS)
  rpt = npad // NS

  hist = pl.kernel(
      functools.partial(_hist_body, chunks=chunks, npad=npad),
      out_type=(jax.ShapeDtypeStruct((NC, npad), jnp.float32),
                jax.ShapeDtypeStruct((NC, npad), jnp.float32)),
      mesh=mesh,
      scratch_types=[
          pltpu.VMEM((chunks, CH), jnp.int32),
          pltpu.VMEM((CH,), jnp.float32),
          pltpu.VMEM((rpt,), jnp.float32),
          pltpu.VMEM_SHARED((npad,), jnp.float32),
          pltpu.VMEM_SHARED((npad,), jnp.float32),
      ],
  )
  degs, degd = hist(src_p, dst_p)

  scale = pl.pallas_call(
      _scale_body,
      grid=(npad // 1024,),
      in_specs=[
          pl.BlockSpec((1024, NC), lambda i: (i, 0)),
          pl.BlockSpec((1024, F), lambda i: (i, 0)),
      ],
      out_specs=pl.BlockSpec((1024, F), lambda i: (i, 0)),
      out_shape=jax.ShapeDtypeStruct((npad, F), jnp.float32),
  )
  feat_scaled = scale(jnp.transpose(degs), feat_pad)

  eidx = jnp.stack([src_p, dst_p], axis=2)
  agg = pl.kernel(
      functools.partial(_agg_body, chunks=chunks, npad=npad),
      out_type=jax.ShapeDtypeStruct((NC, npad, F), jnp.float32),
      mesh=mesh,
      scratch_types=[
          pltpu.VMEM((2, G, 2, CH), jnp.int32),
          pltpu.VMEM((NBUF, CH, F), jnp.float32),
          pltpu.VMEM_SHARED((npad, F), jnp.float32),
          pltpu.SemaphoreType.DMA((NBUF + 2,)),
      ],
  )
  acc = agg(feat_scaled, eidx, jnp.zeros((CH, F), jnp.float32))

  blk = 512
  final = pl.pallas_call(
      _final_body,
      grid=(npad // blk,),
      in_specs=[
          pl.BlockSpec((NC, blk, F), lambda i: (0, i, 0)),
          pl.BlockSpec((blk, NC), lambda i: (i, 0)),
          pl.BlockSpec((blk, F), lambda i: (i, 0)),
          pl.BlockSpec((F, F), lambda i: (0, 0)),
          pl.BlockSpec((F, F), lambda i: (0, 0)),
          pl.BlockSpec((1, F), lambda i: (0, 0)),
      ],
      out_specs=pl.BlockSpec((blk, F), lambda i: (i, 0)),
      out_shape=jax.ShapeDtypeStruct((npad, F), jnp.float32),
  )
  out = final(acc, jnp.transpose(degd), feat_pad, W, W_loop,
              b.reshape(1, F))
  return out[:n]


# speedup vs baseline: 8.8364x; 3.0160x over previous
import functools

import jax
import jax.numpy as jnp
from jax import lax
from jax.experimental import pallas as pl
from jax.experimental.pallas import tpu as pltpu
from jax.experimental.pallas import tpu_sc as plsc

F = 128
NC, NS = 2, 16
NW = NC * NS
CH = 128
LANES = 16


def _hist_body(sidx, didx, degs_out, degd_out, idx_v, ones_v, zero_v, hs, hd,
               *, chunks, npad):
  rpt = npad // NS
  c = lax.axis_index("c")
  s = lax.axis_index("s")
  w = s * NC + c

  def fill(i, _):
    ones_v[pl.ds(i * LANES, LANES)] = jnp.full((LANES,), 1.0, jnp.float32)
    return 0
  lax.fori_loop(0, CH // LANES, fill, 0)

  def zfill(i, _):
    zero_v[pl.ds(i * LANES, LANES)] = jnp.zeros((LANES,), jnp.float32)
    return 0
  lax.fori_loop(0, rpt // LANES, zfill, 0)

  pltpu.sync_copy(zero_v, hs.at[pl.ds(s * rpt, rpt)])
  pltpu.sync_copy(zero_v, hd.at[pl.ds(s * rpt, rpt)])
  plsc.subcore_barrier()

  pltpu.sync_copy(sidx.at[w], idx_v)

  def sbody(j, _):
    pltpu.sync_copy(ones_v, hs.at[idx_v.at[j]], add=True)
    return 0
  lax.fori_loop(0, chunks, sbody, 0)

  pltpu.sync_copy(didx.at[w], idx_v)

  def dbody(j, _):
    pltpu.sync_copy(ones_v, hd.at[idx_v.at[j]], add=True)
    return 0
  lax.fori_loop(0, chunks, dbody, 0)

  plsc.subcore_barrier()
  pltpu.sync_copy(hs.at[pl.ds(s * rpt, rpt)], degs_out.at[c, pl.ds(s * rpt, rpt)])
  pltpu.sync_copy(hd.at[pl.ds(s * rpt, rpt)], degd_out.at[c, pl.ds(s * rpt, rpt)])


NBUF = 2
G = 16


def _agg_body(feat_s, eidx, zrows, acc_out, idx_v, rows, acc, sems,
              *, chunks, npad):
  rpt = npad // NS
  c = lax.axis_index("c")
  s = lax.axis_index("s")
  w = s * NC + c
  ngr = chunks // G

  def load_idx(g):
    return pltpu.async_copy(eidx.at[w, pl.ds(g * G, G)], idx_v.at[g % 2],
                            sems.at[NBUF + g % 2])

  icp0 = load_idx(0)
  pltpu.sync_copy(zrows, rows.at[0])
  for r in range(rpt // CH):
    pltpu.sync_copy(rows.at[0], acc.at[pl.ds(s * rpt + r * CH, CH)])
  icp0.wait()
  plsc.subcore_barrier()

  icps = {}
  idx_ready = {0}
  cps = [pltpu.async_copy(feat_s.at[idx_v.at[0, k, 0]], rows.at[k],
                          sems.at[k]) for k in range(NBUF)]
  for j in range(chunks):
    b = j % NBUF
    if j % G == 0 and j // G + 1 < ngr:
      icps[j // G + 1] = load_idx(j // G + 1)
    cps[b].wait()
    pltpu.sync_copy(rows.at[b], acc.at[idx_v.at[(j // G) % 2, j % G, 1]],
                    add=True)
    nxt = j + NBUF
    if nxt < chunks:
      gi = nxt // G
      if gi not in idx_ready:
        icps[gi].wait()
        idx_ready.add(gi)
      cps[b] = pltpu.async_copy(feat_s.at[idx_v.at[gi % 2, nxt % G, 0]],
                                rows.at[b], sems.at[b])

  plsc.subcore_barrier()
  pltpu.sync_copy(acc.at[pl.ds(s * rpt, rpt)], acc_out.at[c, pl.ds(s * rpt, rpt)])


def _scale_body(deg_ref, feat_ref, out_ref):
  d = deg_ref[:, 0] + deg_ref[:, 1]
  norm = lax.rsqrt(jnp.maximum(d, 1.0))
  out_ref[...] = feat_ref[...] * norm[:, None]


def _final_body(acc_ref, degd_ref, feat_ref, w_ref, wl_ref, b_ref, out_ref):
  agg = acc_ref[0] + acc_ref[1]
  rst = jnp.dot(agg, w_ref[...], preferred_element_type=jnp.float32)
  d = degd_ref[:, 0] + degd_ref[:, 1]
  norm = lax.rsqrt(jnp.maximum(d, 1.0))
  loop_msg = jnp.dot(feat_ref[...], wl_ref[...], preferred_element_type=jnp.float32)
  out_ref[...] = rst * norm[:, None] + b_ref[0] + loop_msg


def kernel(feat, edge_index, W, b, W_loop):
  n = feat.shape[0]
  e = edge_index.shape[1]
  npad = ((n + NS * CH) // (NS * CH)) * (NS * CH)
  grp_e = NW * CH * G
  ept = ((e + grp_e - 1) // grp_e) * (CH * G)
  chunks = ept // CH
  epad = ept * NW

  src = edge_index[0].astype(jnp.int32)
  dst = edge_index[1].astype(jnp.int32)
  trash = n + (jnp.arange(epad - e, dtype=jnp.int32) % (npad - n))
  src_p = jnp.concatenate([src, trash]).reshape(NW, chunks, CH)
  dst_p = jnp.concatenate([dst, trash]).reshape(NW, chunks, CH)
  feat_pad = jnp.pad(feat, ((0, npad - n), (0, 0)))

  mesh = plsc.VectorSubcoreMesh(core_axis_name="c", subcore_axis_name="s",
                                num_cores=NC, num_subcores=NS)
  rpt = npad // NS

  hist = pl.kernel(
      functools.partial(_hist_body, chunks=chunks, npad=npad),
      out_type=(jax.ShapeDtypeStruct((NC, npad), jnp.float32),
                jax.ShapeDtypeStruct((NC, npad), jnp.float32)),
      mesh=mesh,
      scratch_types=[
          pltpu.VMEM((chunks, CH), jnp.int32),
          pltpu.VMEM((CH,), jnp.float32),
          pltpu.VMEM((rpt,), jnp.float32),
          pltpu.VMEM_SHARED((npad,), jnp.float32),
          pltpu.VMEM_SHARED((npad,), jnp.float32),
      ],
  )
  degs, degd = hist(src_p, dst_p)

  scale = pl.pallas_call(
      _scale_body,
      grid=(npad // 1024,),
      in_specs=[
          pl.BlockSpec((1024, NC), lambda i: (i, 0)),
          pl.BlockSpec((1024, F), lambda i: (i, 0)),
      ],
      out_specs=pl.BlockSpec((1024, F), lambda i: (i, 0)),
      out_shape=jax.ShapeDtypeStruct((npad, F), jnp.float32),
  )
  feat_scaled = scale(jnp.transpose(degs), feat_pad)

  eidx = jnp.stack([src_p, dst_p], axis=2)
  agg = pl.kernel(
      functools.partial(_agg_body, chunks=chunks, npad=npad),
      out_type=jax.ShapeDtypeStruct((NC, npad, F), jnp.float32),
      mesh=mesh,
      scratch_types=[
          pltpu.VMEM((2, G, 2, CH), jnp.int32),
          pltpu.VMEM((NBUF, CH, F), jnp.float32),
          pltpu.VMEM_SHARED((npad, F), jnp.float32),
          pltpu.SemaphoreType.DMA((NBUF + 2,)),
      ],
  )
  acc = agg(feat_scaled, eidx, jnp.zeros((CH, F), jnp.float32))

  blk = 512
  final = pl.pallas_call(
      _final_body,
      grid=(npad // blk,),
      in_specs=[
          pl.BlockSpec((NC, blk, F), lambda i: (0, i, 0)),
          pl.BlockSpec((blk, NC), lambda i: (i, 0)),
          pl.BlockSpec((blk, F), lambda i: (i, 0)),
          pl.BlockSpec((F, F), lambda i: (0, 0)),
          pl.BlockSpec((F, F), lambda i: (0, 0)),
          pl.BlockSpec((1, F), lambda i: (0, 0)),
      ],
      out_specs=pl.BlockSpec((blk, F), lambda i: (i, 0)),
      out_shape=jax.ShapeDtypeStruct((npad, F), jnp.float32),
  )
  out = final(acc, jnp.transpose(degd), feat_pad, W, W_loop,
              b.reshape(1, F))
  return out[:n]
